# Initial kernel scaffold; baseline (speedup 1.0000x reference)
import jax
import jax.numpy as jnp
from jax.experimental import pallas as pl


def kernel(x, edge_index, batch, W1, b1, W2, b2, W3, b3, Wl, bl):
    raise NotImplementedError("write your pallas kernel here")


# speedup vs baseline: 10.4968x; 10.4968x over previous
import jax
import jax.numpy as jnp
from jax import lax
from jax.experimental import pallas as pl
from jax.experimental.pallas import tpu as pltpu
from jax.experimental.pallas import tpu_sc as plsc

NC, NS, L = 2, 16, 16
NW = NC * NS
K = 128
BCH = 8
ZR = 112
NB = 1792


def _rup(x, m):
    return (x + m - 1) // m * m


def _agg_call(gflat, src2d, dst2d, *, n_pad, e_pad, dc, n_slab_sc,
              split_edges, interpret=False):
    n_out = NC if split_edges else NC * n_slab_sc
    stripe = n_pad // NS

    def body(gflat_ref, src_ref, dst_ref, out_ref,
             acc, srcb, dstb, rows0, rows1, zbuf, sem0, sem1):
        c = lax.axis_index("c")
        s = lax.axis_index("s")
        wid = c * NS + s

        def zb(i, carry):
            for j in range(dc // L):
                zbuf[i, pl.ds(j * L, L)] = jnp.zeros((L,), jnp.float32)
            return carry
        lax.fori_loop(0, ZR, zb, 0)

        for phase in range(n_slab_sc):
            def zc(z, carry):
                st = pl.multiple_of(s * stripe + z * ZR, 8)
                pltpu.sync_copy(zbuf, acc.at[pl.ds(st, ZR)])
                return carry
            lax.fori_loop(0, stripe // ZR, zc, 0)
            plsc.subcore_barrier()

            if split_edges:
                ept = e_pad // NW
                ebase = wid * ept
                off = None
            else:
                ept = e_pad // NS
                ebase = s * ept
                off = (c * n_slab_sc + phase) * n_pad

            def blk(b, carry):
                cb = pl.multiple_of(ebase // K + b * BCH, 8)
                pltpu.sync_copy(src_ref.at[pl.ds(cb, BCH)], srcb)
                pltpu.sync_copy(dst_ref.at[pl.ds(cb, BCH)], dstb)
                if off is not None:
                    for r in range(BCH):
                        for i in range(K // L):
                            sl = pl.ds(i * L, L)
                            srcb[r, sl] = srcb[r, sl] + off
                for r in range(BCH):
                    rows = rows0 if r % 2 == 0 else rows1
                    sem = sem0 if r % 2 == 0 else sem1
                    pltpu.async_copy(gflat_ref.at[srcb.at[r]], rows, sem).wait()
                    pltpu.sync_copy(rows, acc.at[dstb.at[r]], add=True)
                return carry
            lax.fori_loop(0, ept // (BCH * K), blk, 0)
            plsc.subcore_barrier()

            oi = c if split_edges else c * n_slab_sc + phase
            st = pl.multiple_of(s * stripe, 8)
            pltpu.sync_copy(acc.at[pl.ds(st, stripe)],
                            out_ref.at[oi, pl.ds(st, stripe)])
            if phase + 1 < n_slab_sc:
                plsc.subcore_barrier()

    fn = pl.kernel(
        body,
        out_type=jax.ShapeDtypeStruct((n_out, n_pad, dc), jnp.float32),
        mesh=plsc.VectorSubcoreMesh(core_axis_name="c", subcore_axis_name="s", num_cores=NC, num_subcores=NS),
        compiler_params=pltpu.CompilerParams(use_tc_tiling_on_sc=False),
        scratch_types=[
            pltpu.VMEM_SHARED((n_pad, dc), jnp.float32),
            pltpu.VMEM((BCH, K), jnp.int32),
            pltpu.VMEM((BCH, K), jnp.int32),
            pltpu.VMEM((K, dc), jnp.float32),
            pltpu.VMEM((K, dc), jnp.float32),
            pltpu.VMEM((ZR, dc), jnp.float32),
            pltpu.SemaphoreType.DMA,
            pltpu.SemaphoreType.DMA,
        ],
        interpret=interpret,
    )
    return fn(gflat, src2d, dst2d)


def _deg_call(dst2d, *, n_pad, e_pad, interpret=False):
    dc = 16
    stripe = n_pad // NS

    def body(dst_ref, out_ref, acc, dstb, ones_rows, zbuf):
        c = lax.axis_index("c")
        s = lax.axis_index("s")
        wid = c * NS + s

        def zb(i, carry):
            zbuf[i, pl.ds(0, L)] = jnp.zeros((L,), jnp.float32)
            return carry
        lax.fori_loop(0, ZR, zb, 0)

        def ob(i, carry):
            ones_rows[i, pl.ds(0, L)] = jnp.ones((L,), jnp.float32)
            return carry
        lax.fori_loop(0, K, ob, 0)

        def zc(z, carry):
            st = pl.multiple_of(s * stripe + z * ZR, 8)
            pltpu.sync_copy(zbuf, acc.at[pl.ds(st, ZR)])
            return carry
        lax.fori_loop(0, stripe // ZR, zc, 0)
        plsc.subcore_barrier()

        ept = e_pad // NW
        ebase = wid * ept

        def blk(b, carry):
            cb = pl.multiple_of(ebase // K + b * BCH, 8)
            pltpu.sync_copy(dst_ref.at[pl.ds(cb, BCH)], dstb)
            for r in range(BCH):
                pltpu.sync_copy(ones_rows, acc.at[dstb.at[r]], add=True)
            return carry
        lax.fori_loop(0, ept // (BCH * K), blk, 0)
        plsc.subcore_barrier()

        st = pl.multiple_of(s * stripe, 8)
        pltpu.sync_copy(acc.at[pl.ds(st, stripe)],
                        out_ref.at[c, pl.ds(st, stripe)])

    fn = pl.kernel(
        body,
        out_type=jax.ShapeDtypeStruct((NC, n_pad, dc), jnp.float32),
        mesh=plsc.VectorSubcoreMesh(core_axis_name="c", subcore_axis_name="s", num_cores=NC, num_subcores=NS),
        compiler_params=pltpu.CompilerParams(use_tc_tiling_on_sc=False),
        scratch_types=[
            pltpu.VMEM_SHARED((n_pad, dc), jnp.float32),
            pltpu.VMEM((BCH, K), jnp.int32),
            pltpu.VMEM((K, dc), jnp.float32),
            pltpu.VMEM((ZR, dc), jnp.float32),
        ],
        interpret=interpret,
    )
    return fn(dst2d)


def _pool_call(h3, bat32, *, n_pad, g_pad, dc, interpret=False):
    npt = n_pad // NW
    nj = npt // 32
    gs = g_pad // NS

    def body(h3_ref, bat_ref, out_ref, pacc, rbuf, batb, zbuf):
        c = lax.axis_index("c")
        s = lax.axis_index("s")
        wid = c * NS + s

        def zb(i, carry):
            for j in range(dc // L):
                zbuf[i, pl.ds(j * L, L)] = jnp.zeros((L,), jnp.float32)
            return carry
        lax.fori_loop(0, gs, zb, 0)
        st = pl.multiple_of(s * gs, 8)
        pltpu.sync_copy(zbuf, pacc.at[pl.ds(st, gs)])
        plsc.subcore_barrier()

        pltpu.sync_copy(bat_ref.at[wid], batb)

        def jl(j, carry):
            hb = pl.multiple_of(wid * npt + j * 32, 8)
            pltpu.sync_copy(h3_ref.at[pl.ds(hb, 32)], rbuf)
            pltpu.sync_copy(rbuf, pacc.at[batb.at[j]], add=True)
            return carry
        lax.fori_loop(0, nj, jl, 0)
        plsc.subcore_barrier()

        pltpu.sync_copy(pacc.at[pl.ds(st, gs)],
                        out_ref.at[c, pl.ds(st, gs)])

    fn = pl.kernel(
        body,
        out_type=jax.ShapeDtypeStruct((NC, g_pad, dc), jnp.float32),
        mesh=plsc.VectorSubcoreMesh(core_axis_name="c", subcore_axis_name="s", num_cores=NC, num_subcores=NS),
        compiler_params=pltpu.CompilerParams(use_tc_tiling_on_sc=False),
        scratch_types=[
            pltpu.VMEM_SHARED((g_pad, dc), jnp.float32),
            pltpu.VMEM((32, dc), jnp.float32),
            pltpu.VMEM((nj, 32), jnp.int32),
            pltpu.VMEM((gs, dc), jnp.float32),
        ],
        interpret=interpret,
    )
    return fn(h3, bat32)


def _tc0(degp, xp, *, n_pad, interpret=False):
    def body(degp_ref, xp_ref, dinv_ref, g0_ref):
        deg = degp_ref[0, :, 0:1] + degp_ref[1, :, 0:1] + 1.0
        di = lax.rsqrt(deg)
        dinv_ref[...] = di
        g0_ref[...] = di * xp_ref[...]

    return pl.pallas_call(
        body,
        grid=(n_pad // NB,),
        in_specs=[pl.BlockSpec((NC, NB, 16), lambda i: (0, i, 0)),
                  pl.BlockSpec((NB, 16), lambda i: (i, 0))],
        out_specs=[pl.BlockSpec((NB, 1), lambda i: (i, 0)),
                   pl.BlockSpec((NB, 16), lambda i: (i, 0))],
        out_shape=[jax.ShapeDtypeStruct((n_pad, 1), jnp.float32),
                   jax.ShapeDtypeStruct((n_pad, 16), jnp.float32)],
        interpret=interpret,
    )(degp, xp)


def _tc1(agg0, g0, dinv, W1p, b1, *, n_pad, interpret=False):
    def body(a_ref, g_ref, d_ref, w_ref, b_ref, o_ref):
        z = d_ref[...] * (a_ref[0] + a_ref[1] + g_ref[...])
        h = jnp.dot(z.astype(jnp.bfloat16), w_ref[...].astype(jnp.bfloat16),
                    preferred_element_type=jnp.float32)
        h = jnp.maximum(h + b_ref[...], 0.0)
        g1 = d_ref[...] * h
        for j in range(4):
            o_ref[j] = g1[:, j * 32:(j + 1) * 32]

    return pl.pallas_call(
        body,
        grid=(n_pad // NB,),
        in_specs=[pl.BlockSpec((NC, NB, 16), lambda i: (0, i, 0)),
                  pl.BlockSpec((NB, 16), lambda i: (i, 0)),
                  pl.BlockSpec((NB, 1), lambda i: (i, 0)),
                  pl.BlockSpec((16, 128), lambda i: (0, 0)),
                  pl.BlockSpec((1, 128), lambda i: (0, 0))],
        out_specs=pl.BlockSpec((4, NB, 32), lambda i: (0, i, 0)),
        out_shape=jax.ShapeDtypeStruct((4, n_pad, 32), jnp.float32),
        interpret=interpret,
    )(agg0, g0, dinv, W1p, b1)


def _tc2(agg1, g1, dinv, W2, b2, W3, *, n_pad, interpret=False):
    def body(a_ref, g_ref, d_ref, w2_ref, b2_ref, w3_ref, o_ref):
        zs = a_ref[...] + g_ref[...]
        z = jnp.concatenate([zs[j] for j in range(4)], axis=1)
        z = d_ref[...] * z
        h2 = jnp.dot(z.astype(jnp.bfloat16), w2_ref[...].astype(jnp.bfloat16),
                     preferred_element_type=jnp.float32)
        h2 = jnp.maximum(h2 + b2_ref[...], 0.0)
        t = jnp.dot(h2.astype(jnp.bfloat16), w3_ref[...].astype(jnp.bfloat16),
                    preferred_element_type=jnp.float32)
        g2 = d_ref[...] * t
        for j in range(2):
            o_ref[j] = g2[:, j * 32:(j + 1) * 32]

    return pl.pallas_call(
        body,
        grid=(n_pad // NB,),
        in_specs=[pl.BlockSpec((4, NB, 32), lambda i: (0, i, 0)),
                  pl.BlockSpec((4, NB, 32), lambda i: (0, i, 0)),
                  pl.BlockSpec((NB, 1), lambda i: (i, 0)),
                  pl.BlockSpec((128, 128), lambda i: (0, 0)),
                  pl.BlockSpec((1, 128), lambda i: (0, 0)),
                  pl.BlockSpec((128, 64), lambda i: (0, 0))],
        out_specs=pl.BlockSpec((2, NB, 32), lambda i: (0, i, 0)),
        out_shape=jax.ShapeDtypeStruct((2, n_pad, 32), jnp.float32),
        interpret=interpret,
    )(agg1, g1, dinv, W2, b2, W3)


def _tc3(agg2, g2, dinv, b3, *, n_pad, interpret=False):
    def body(a_ref, g_ref, d_ref, b_ref, o_ref):
        zs = a_ref[...] + g_ref[...]
        z = jnp.concatenate([zs[0], zs[1]], axis=1)
        h3 = jnp.maximum(d_ref[...] * z + b_ref[...], 0.0)
        o_ref[...] = jnp.concatenate(
            [h3, jnp.ones((h3.shape[0], 16), jnp.float32)], axis=1)

    return pl.pallas_call(
        body,
        grid=(n_pad // NB,),
        in_specs=[pl.BlockSpec((2, NB, 32), lambda i: (0, i, 0)),
                  pl.BlockSpec((2, NB, 32), lambda i: (0, i, 0)),
                  pl.BlockSpec((NB, 1), lambda i: (i, 0)),
                  pl.BlockSpec((1, 64), lambda i: (0, 0))],
        out_specs=pl.BlockSpec((NB, 80), lambda i: (i, 0)),
        out_shape=jax.ShapeDtypeStruct((n_pad, 80), jnp.float32),
        interpret=interpret,
    )(agg2, g2, dinv, b3)


def _tc4(poolp, Wl, bl, *, g_pad, interpret=False):
    def body(p_ref, wl_ref, bl_ref, o_ref):
        pp = p_ref[0] + p_ref[1]
        sm = pp[:, :64]
        cnt = pp[:, 64:65]
        y = jnp.dot(sm.astype(jnp.bfloat16), wl_ref[...].astype(jnp.bfloat16),
                    preferred_element_type=jnp.float32)
        o_ref[...] = y / jnp.maximum(cnt, 1.0) + bl_ref[...]

    return pl.pallas_call(
        body,
        out_shape=jax.ShapeDtypeStruct((g_pad, 1), jnp.float32),
        interpret=interpret,
    )(poolp, Wl, bl)


def _run(x, edge_index, batch, W1, b1, W2, b2, W3, b3, Wl, bl,
         interpret=False):
    f32 = jnp.float32
    N = x.shape[0]
    E = edge_index.shape[1]
    G = 512

    n_pad = _rup(N + 1, 7168)
    e_pad = _rup(E, NW * BCH * K)
    g_pad = _rup(G + 1, 128)

    src = edge_index[0].astype(jnp.int32)
    dst = edge_index[1].astype(jnp.int32)
    src_p = jnp.concatenate([src, jnp.zeros((e_pad - E,), jnp.int32)])
    dst_p = jnp.concatenate([dst, jnp.full((e_pad - E,), N, jnp.int32)])
    src2d = src_p.reshape(e_pad // K, K)
    dst2d = dst_p.reshape(e_pad // K, K)
    bat = batch.astype(jnp.int32)
    bat_p = jnp.concatenate([bat, jnp.full((n_pad - N,), G, jnp.int32)])
    bat32 = bat_p.reshape(NW, (n_pad // NW) // 32, 32)
    xp = jnp.zeros((n_pad, 16), f32).at[:N, :3].set(x)
    W1p = jnp.zeros((16, 128), f32).at[:3].set(W1)
    b1r = b1.reshape(1, 128)
    b2r = b2.reshape(1, 128)
    b3r = b3.reshape(1, 64)
    blr = bl.reshape(1, 1)
    wlr = Wl.reshape(64, 1)

    degp = _deg_call(dst2d, n_pad=n_pad, e_pad=e_pad, interpret=interpret)
    dinv, g0 = _tc0(degp, xp, n_pad=n_pad, interpret=interpret)
    agg0 = _agg_call(g0, src2d, dst2d, n_pad=n_pad, e_pad=e_pad, dc=16,
                     n_slab_sc=1, split_edges=True, interpret=interpret)
    g1 = _tc1(agg0, g0, dinv, W1p, b1r, n_pad=n_pad, interpret=interpret)
    agg1 = _agg_call(g1.reshape(4 * n_pad, 32), src2d, dst2d, n_pad=n_pad,
                     e_pad=e_pad, dc=32, n_slab_sc=2, split_edges=False,
                     interpret=interpret)
    g2 = _tc2(agg1, g1, dinv, W2, b2r, W3, n_pad=n_pad, interpret=interpret)
    agg2 = _agg_call(g2.reshape(2 * n_pad, 32), src2d, dst2d, n_pad=n_pad,
                     e_pad=e_pad, dc=32, n_slab_sc=1, split_edges=False,
                     interpret=interpret)
    h3 = _tc3(agg2, g2, dinv, b3r, n_pad=n_pad, interpret=interpret)
    poolp = _pool_call(h3, bat32, n_pad=n_pad, g_pad=g_pad, dc=80,
                       interpret=interpret)
    out = _tc4(poolp, wlr, blr, g_pad=g_pad, interpret=interpret)
    return out[:G]


def kernel(x, edge_index, batch, W1, b1, W2, b2, W3, b3, Wl, bl):
    return _run(x, edge_index, batch, W1, b1, W2, b2, W3, b3, Wl, bl)


# speedup vs baseline: 13.7047x; 1.3056x over previous
import jax
import jax.numpy as jnp
from jax import lax
from jax.experimental import pallas as pl
from jax.experimental.pallas import tpu as pltpu
from jax.experimental.pallas import tpu_sc as plsc

NC, NS, L = 2, 16, 16
NW = NC * NS
K = 128
BCH = 8
ZR = 112
NB = 1792


def _rup(x, m):
    return (x + m - 1) // m * m


def _agg_call(gflat, src2d, dst2d, *, n_pad, e_pad, dc, n_slab_sc,
              split_edges, interpret=False):
    n_out = NC if split_edges else NC * n_slab_sc
    stripe = n_pad // NS

    def body(gflat_ref, src_ref, dst_ref, out_ref,
             acc, srcb, dstb, rows, zbuf, gsem, ssem):
        c = lax.axis_index("c")
        s = lax.axis_index("s")
        wid = c * NS + s

        def zb(i, carry):
            for j in range(dc // L):
                zbuf[i, pl.ds(j * L, L)] = jnp.zeros((L,), jnp.float32)
            return carry
        lax.fori_loop(0, ZR, zb, 0)

        for phase in range(n_slab_sc):
            def zc(z, carry):
                st = pl.multiple_of(s * stripe + z * ZR, 8)
                pltpu.sync_copy(zbuf, acc.at[pl.ds(st, ZR)])
                return carry
            lax.fori_loop(0, stripe // ZR, zc, 0)
            plsc.subcore_barrier()

            if split_edges:
                ept = e_pad // NW
                ebase = wid * ept
                off = None
            else:
                ept = e_pad // NS
                ebase = s * ept
                off = (c * n_slab_sc + phase) * n_pad

            def blk(b, carry):
                bank = b % 2
                cb = pl.multiple_of(ebase // K + b * BCH, 8)
                pltpu.sync_copy(src_ref.at[pl.ds(cb, BCH)], srcb)
                pltpu.sync_copy(dst_ref.at[pl.ds(cb, BCH)], dstb.at[bank])
                if off is not None:
                    for r in range(BCH):
                        for i in range(K // L):
                            sl = pl.ds(i * L, L)
                            srcb[r, sl] = srcb[r, sl] + off
                for g in range(2):
                    def _drain():
                        for i in range(4):
                            pltpu.make_async_copy(
                                rows.at[i], acc.at[pl.ds(0, K)],
                                ssem).wait()
                    if g == 1:
                        _drain()
                    else:
                        pl.when(b >= 1)(_drain)
                    gd = []
                    for i in range(4):
                        gd.append(pltpu.async_copy(
                            gflat_ref.at[srcb.at[g * 4 + i]],
                            rows.at[i], gsem.at[i]))
                    for i in range(4):
                        gd[i].wait()
                        pltpu.async_copy(rows.at[i],
                                         acc.at[dstb.at[bank, g * 4 + i]],
                                         ssem, add=True)
                return carry
            lax.fori_loop(0, ept // (BCH * K), blk, 0)
            for i in range(4):
                pltpu.make_async_copy(rows.at[i], acc.at[pl.ds(0, K)],
                                      ssem).wait()
            plsc.subcore_barrier()

            oi = c if split_edges else c * n_slab_sc + phase
            st = pl.multiple_of(s * stripe, 8)
            pltpu.sync_copy(acc.at[pl.ds(st, stripe)],
                            out_ref.at[oi, pl.ds(st, stripe)])
            if phase + 1 < n_slab_sc:
                plsc.subcore_barrier()

    fn = pl.kernel(
        body,
        out_type=jax.ShapeDtypeStruct((n_out, n_pad, dc), jnp.float32),
        mesh=plsc.VectorSubcoreMesh(core_axis_name="c", subcore_axis_name="s", num_cores=NC, num_subcores=NS),
        compiler_params=pltpu.CompilerParams(use_tc_tiling_on_sc=False),
        scratch_types=[
            pltpu.VMEM_SHARED((n_pad, dc), jnp.float32),
            pltpu.VMEM((BCH, K), jnp.int32),
            pltpu.VMEM((2, BCH, K), jnp.int32),
            pltpu.VMEM((4, K, dc), jnp.float32),
            pltpu.VMEM((ZR, dc), jnp.float32),
            pltpu.SemaphoreType.DMA((4,)),
            pltpu.SemaphoreType.DMA,
        ],
        interpret=interpret,
    )
    return fn(gflat, src2d, dst2d)


def _deg_call(dst2d, *, n_pad, e_pad, interpret=False):
    dc = 16
    stripe = n_pad // NS

    def body(dst_ref, out_ref, acc, dstb, ones_rows, zbuf):
        c = lax.axis_index("c")
        s = lax.axis_index("s")
        wid = c * NS + s

        def zb(i, carry):
            zbuf[i, pl.ds(0, L)] = jnp.zeros((L,), jnp.float32)
            return carry
        lax.fori_loop(0, ZR, zb, 0)

        def ob(i, carry):
            ones_rows[i, pl.ds(0, L)] = jnp.ones((L,), jnp.float32)
            return carry
        lax.fori_loop(0, K, ob, 0)

        def zc(z, carry):
            st = pl.multiple_of(s * stripe + z * ZR, 8)
            pltpu.sync_copy(zbuf, acc.at[pl.ds(st, ZR)])
            return carry
        lax.fori_loop(0, stripe // ZR, zc, 0)
        plsc.subcore_barrier()

        ept = e_pad // NW
        ebase = wid * ept

        def blk(b, carry):
            cb = pl.multiple_of(ebase // K + b * BCH, 8)
            pltpu.sync_copy(dst_ref.at[pl.ds(cb, BCH)], dstb)
            for r in range(BCH):
                pltpu.sync_copy(ones_rows, acc.at[dstb.at[r]], add=True)
            return carry
        lax.fori_loop(0, ept // (BCH * K), blk, 0)
        plsc.subcore_barrier()

        st = pl.multiple_of(s * stripe, 8)
        pltpu.sync_copy(acc.at[pl.ds(st, stripe)],
                        out_ref.at[c, pl.ds(st, stripe)])

    fn = pl.kernel(
        body,
        out_type=jax.ShapeDtypeStruct((NC, n_pad, dc), jnp.float32),
        mesh=plsc.VectorSubcoreMesh(core_axis_name="c", subcore_axis_name="s", num_cores=NC, num_subcores=NS),
        compiler_params=pltpu.CompilerParams(use_tc_tiling_on_sc=False),
        scratch_types=[
            pltpu.VMEM_SHARED((n_pad, dc), jnp.float32),
            pltpu.VMEM((BCH, K), jnp.int32),
            pltpu.VMEM((K, dc), jnp.float32),
            pltpu.VMEM((ZR, dc), jnp.float32),
        ],
        interpret=interpret,
    )
    return fn(dst2d)


def _pool_call(h3, bat32, *, n_pad, g_pad, dc, interpret=False):
    npt = n_pad // NW
    nj = npt // 32
    gs = g_pad // NS

    def body(h3_ref, bat_ref, out_ref, pacc, rbuf, batb, zbuf):
        c = lax.axis_index("c")
        s = lax.axis_index("s")
        wid = c * NS + s

        def zb(i, carry):
            for j in range(dc // L):
                zbuf[i, pl.ds(j * L, L)] = jnp.zeros((L,), jnp.float32)
            return carry
        lax.fori_loop(0, gs, zb, 0)
        st = pl.multiple_of(s * gs, 8)
        pltpu.sync_copy(zbuf, pacc.at[pl.ds(st, gs)])
        plsc.subcore_barrier()

        pltpu.sync_copy(bat_ref.at[wid], batb)

        def jl(j, carry):
            hb = pl.multiple_of(wid * npt + j * 32, 8)
            pltpu.sync_copy(h3_ref.at[pl.ds(hb, 32)], rbuf)
            pltpu.sync_copy(rbuf, pacc.at[batb.at[j]], add=True)
            return carry
        lax.fori_loop(0, nj, jl, 0)
        plsc.subcore_barrier()

        pltpu.sync_copy(pacc.at[pl.ds(st, gs)],
                        out_ref.at[c, pl.ds(st, gs)])

    fn = pl.kernel(
        body,
        out_type=jax.ShapeDtypeStruct((NC, g_pad, dc), jnp.float32),
        mesh=plsc.VectorSubcoreMesh(core_axis_name="c", subcore_axis_name="s", num_cores=NC, num_subcores=NS),
        compiler_params=pltpu.CompilerParams(use_tc_tiling_on_sc=False),
        scratch_types=[
            pltpu.VMEM_SHARED((g_pad, dc), jnp.float32),
            pltpu.VMEM((32, dc), jnp.float32),
            pltpu.VMEM((nj, 32), jnp.int32),
            pltpu.VMEM((gs, dc), jnp.float32),
        ],
        interpret=interpret,
    )
    return fn(h3, bat32)


def _tc0(degp, xp, *, n_pad, interpret=False):
    def body(degp_ref, xp_ref, dinv_ref, g0_ref):
        deg = degp_ref[0, :, 0:1] + degp_ref[1, :, 0:1] + 1.0
        di = lax.rsqrt(deg)
        dinv_ref[...] = di
        g0_ref[...] = di * xp_ref[...]

    return pl.pallas_call(
        body,
        grid=(n_pad // NB,),
        in_specs=[pl.BlockSpec((NC, NB, 16), lambda i: (0, i, 0)),
                  pl.BlockSpec((NB, 16), lambda i: (i, 0))],
        out_specs=[pl.BlockSpec((NB, 1), lambda i: (i, 0)),
                   pl.BlockSpec((NB, 16), lambda i: (i, 0))],
        out_shape=[jax.ShapeDtypeStruct((n_pad, 1), jnp.float32),
                   jax.ShapeDtypeStruct((n_pad, 16), jnp.float32)],
        interpret=interpret,
    )(degp, xp)


def _tc1(agg0, g0, dinv, W1p, b1, *, n_pad, interpret=False):
    def body(a_ref, g_ref, d_ref, w_ref, b_ref, o_ref):
        z = d_ref[...] * (a_ref[0] + a_ref[1] + g_ref[...])
        h = jnp.dot(z.astype(jnp.bfloat16), w_ref[...].astype(jnp.bfloat16),
                    preferred_element_type=jnp.float32)
        h = jnp.maximum(h + b_ref[...], 0.0)
        g1 = d_ref[...] * h
        for j in range(4):
            o_ref[j] = g1[:, j * 32:(j + 1) * 32]

    return pl.pallas_call(
        body,
        grid=(n_pad // NB,),
        in_specs=[pl.BlockSpec((NC, NB, 16), lambda i: (0, i, 0)),
                  pl.BlockSpec((NB, 16), lambda i: (i, 0)),
                  pl.BlockSpec((NB, 1), lambda i: (i, 0)),
                  pl.BlockSpec((16, 128), lambda i: (0, 0)),
                  pl.BlockSpec((1, 128), lambda i: (0, 0))],
        out_specs=pl.BlockSpec((4, NB, 32), lambda i: (0, i, 0)),
        out_shape=jax.ShapeDtypeStruct((4, n_pad, 32), jnp.float32),
        interpret=interpret,
    )(agg0, g0, dinv, W1p, b1)


def _tc2(agg1, g1, dinv, W2, b2, W3, *, n_pad, interpret=False):
    def body(a_ref, g_ref, d_ref, w2_ref, b2_ref, w3_ref, o_ref):
        zs = a_ref[...] + g_ref[...]
        z = jnp.concatenate([zs[j] for j in range(4)], axis=1)
        z = d_ref[...] * z
        h2 = jnp.dot(z.astype(jnp.bfloat16), w2_ref[...].astype(jnp.bfloat16),
                     preferred_element_type=jnp.float32)
        h2 = jnp.maximum(h2 + b2_ref[...], 0.0)
        t = jnp.dot(h2.astype(jnp.bfloat16), w3_ref[...].astype(jnp.bfloat16),
                    preferred_element_type=jnp.float32)
        g2 = d_ref[...] * t
        for j in range(2):
            o_ref[j] = g2[:, j * 32:(j + 1) * 32]

    return pl.pallas_call(
        body,
        grid=(n_pad // NB,),
        in_specs=[pl.BlockSpec((4, NB, 32), lambda i: (0, i, 0)),
                  pl.BlockSpec((4, NB, 32), lambda i: (0, i, 0)),
                  pl.BlockSpec((NB, 1), lambda i: (i, 0)),
                  pl.BlockSpec((128, 128), lambda i: (0, 0)),
                  pl.BlockSpec((1, 128), lambda i: (0, 0)),
                  pl.BlockSpec((128, 64), lambda i: (0, 0))],
        out_specs=pl.BlockSpec((2, NB, 32), lambda i: (0, i, 0)),
        out_shape=jax.ShapeDtypeStruct((2, n_pad, 32), jnp.float32),
        interpret=interpret,
    )(agg1, g1, dinv, W2, b2, W3)


def _tc3(agg2, g2, dinv, b3, *, n_pad, interpret=False):
    def body(a_ref, g_ref, d_ref, b_ref, o_ref):
        zs = a_ref[...] + g_ref[...]
        z = jnp.concatenate([zs[0], zs[1]], axis=1)
        h3 = jnp.maximum(d_ref[...] * z + b_ref[...], 0.0)
        o_ref[...] = jnp.concatenate(
            [h3, jnp.ones((h3.shape[0], 16), jnp.float32)], axis=1)

    return pl.pallas_call(
        body,
        grid=(n_pad // NB,),
        in_specs=[pl.BlockSpec((2, NB, 32), lambda i: (0, i, 0)),
                  pl.BlockSpec((2, NB, 32), lambda i: (0, i, 0)),
                  pl.BlockSpec((NB, 1), lambda i: (i, 0)),
                  pl.BlockSpec((1, 64), lambda i: (0, 0))],
        out_specs=pl.BlockSpec((NB, 80), lambda i: (i, 0)),
        out_shape=jax.ShapeDtypeStruct((n_pad, 80), jnp.float32),
        interpret=interpret,
    )(agg2, g2, dinv, b3)


def _tc4(poolp, Wl, bl, *, g_pad, interpret=False):
    def body(p_ref, wl_ref, bl_ref, o_ref):
        pp = p_ref[0] + p_ref[1]
        sm = pp[:, :64]
        cnt = pp[:, 64:65]
        y = jnp.dot(sm.astype(jnp.bfloat16), wl_ref[...].astype(jnp.bfloat16),
                    preferred_element_type=jnp.float32)
        o_ref[...] = y / jnp.maximum(cnt, 1.0) + bl_ref[...]

    return pl.pallas_call(
        body,
        out_shape=jax.ShapeDtypeStruct((g_pad, 1), jnp.float32),
        interpret=interpret,
    )(poolp, Wl, bl)


def _run(x, edge_index, batch, W1, b1, W2, b2, W3, b3, Wl, bl,
         interpret=False):
    f32 = jnp.float32
    N = x.shape[0]
    E = edge_index.shape[1]
    G = 512

    n_pad = _rup(N + 1, 7168)
    e_pad = _rup(E, NW * BCH * K)
    g_pad = _rup(G + 1, 128)

    src = edge_index[0].astype(jnp.int32)
    dst = edge_index[1].astype(jnp.int32)
    src_p = jnp.concatenate([src, jnp.zeros((e_pad - E,), jnp.int32)])
    dst_p = jnp.concatenate([dst, jnp.full((e_pad - E,), N, jnp.int32)])
    src2d = src_p.reshape(e_pad // K, K)
    dst2d = dst_p.reshape(e_pad // K, K)
    bat = batch.astype(jnp.int32)
    bat_p = jnp.concatenate([bat, jnp.full((n_pad - N,), G, jnp.int32)])
    bat32 = bat_p.reshape(NW, (n_pad // NW) // 32, 32)
    xp = jnp.zeros((n_pad, 16), f32).at[:N, :3].set(x)
    W1p = jnp.zeros((16, 128), f32).at[:3].set(W1)
    b1r = b1.reshape(1, 128)
    b2r = b2.reshape(1, 128)
    b3r = b3.reshape(1, 64)
    blr = bl.reshape(1, 1)
    wlr = Wl.reshape(64, 1)

    degp = _deg_call(dst2d, n_pad=n_pad, e_pad=e_pad, interpret=interpret)
    dinv, g0 = _tc0(degp, xp, n_pad=n_pad, interpret=interpret)
    agg0 = _agg_call(g0, src2d, dst2d, n_pad=n_pad, e_pad=e_pad, dc=16,
                     n_slab_sc=1, split_edges=True, interpret=interpret)
    g1 = _tc1(agg0, g0, dinv, W1p, b1r, n_pad=n_pad, interpret=interpret)
    agg1 = _agg_call(g1.reshape(4 * n_pad, 32), src2d, dst2d, n_pad=n_pad,
                     e_pad=e_pad, dc=32, n_slab_sc=2, split_edges=False,
                     interpret=interpret)
    g2 = _tc2(agg1, g1, dinv, W2, b2r, W3, n_pad=n_pad, interpret=interpret)
    agg2 = _agg_call(g2.reshape(2 * n_pad, 32), src2d, dst2d, n_pad=n_pad,
                     e_pad=e_pad, dc=32, n_slab_sc=1, split_edges=False,
                     interpret=interpret)
    h3 = _tc3(agg2, g2, dinv, b3r, n_pad=n_pad, interpret=interpret)
    poolp = _pool_call(h3, bat32, n_pad=n_pad, g_pad=g_pad, dc=80,
                       interpret=interpret)
    out = _tc4(poolp, wlr, blr, g_pad=g_pad, interpret=interpret)
    return out[:G]


def kernel(x, edge_index, batch, W1, b1, W2, b2, W3, b3, Wl, bl):
    return _run(x, edge_index, batch, W1, b1, W2, b2, W3, b3, Wl, bl)


# speedup vs baseline: 14.5161x; 1.0592x over previous
import jax
import jax.numpy as jnp
from jax import lax
from jax.experimental import pallas as pl
from jax.experimental.pallas import tpu as pltpu
from jax.experimental.pallas import tpu_sc as plsc

NC, NS, L = 2, 16, 16
NW = NC * NS
K = 128
BCH = 8
ZR = 112
NB = 1792

_SC_MESH = dict(core_axis_name="c", subcore_axis_name="s",
                num_cores=NC, num_subcores=NS)
_SC_PARAMS = dict(use_tc_tiling_on_sc=False)


def _rup(x, m):
    return (x + m - 1) // m * m


def _agg_call(gflat, src2d, dst2d, *, n_pad, e_pad, dc, n_slab_sc,
              split_edges, interpret=False):
    mul = 128 // dc
    stripe = n_pad // NS

    def body(gflat_ref, src_ref, dst_ref, out_ref,
             acc, srcb, dstb, rows, zbuf, gsem, ssem):
        c = lax.axis_index("c")
        s = lax.axis_index("s")
        wid = c * NS + s

        def zb(i, carry):
            for j in range(dc // L):
                zbuf[i, pl.ds(j * L, L)] = jnp.zeros((L,), jnp.float32)
            return carry
        lax.fori_loop(0, ZR, zb, 0)

        for phase in range(n_slab_sc):
            def zc(z, carry):
                st = pl.multiple_of(s * stripe + z * ZR, 8)
                pltpu.sync_copy(zbuf, acc.at[pl.ds(st, ZR)])
                return carry
            lax.fori_loop(0, stripe // ZR, zc, 0)
            plsc.subcore_barrier()

            if split_edges:
                ept = e_pad // NW
                ebase = wid * ept
                oslab = c
                gslab = None
            else:
                ept = e_pad // NS
                ebase = s * ept
                oslab = c * n_slab_sc + phase
                gslab = oslab

            def blk(b, carry):
                bank = b % 2
                cb = pl.multiple_of(ebase // K + b * BCH, 8)
                pltpu.sync_copy(src_ref.at[pl.ds(cb, BCH)], srcb)
                pltpu.sync_copy(dst_ref.at[pl.ds(cb, BCH)], dstb.at[bank])
                for r in range(BCH):
                    for i in range(K // L):
                        sl = pl.ds(i * L, L)
                        v = srcb[r, sl] * mul
                        if gslab is not None:
                            v = v + gslab
                        srcb[r, sl] = v
                for g in range(2):
                    def _drain():
                        for i in range(4):
                            pltpu.make_async_copy(
                                rows.at[i], acc.at[pl.ds(0, K)],
                                ssem).wait()
                    if g == 1:
                        _drain()
                    else:
                        pl.when(b >= 1)(_drain)
                    gd = []
                    for i in range(4):
                        gd.append(pltpu.async_copy(
                            gflat_ref.at[srcb.at[g * 4 + i]],
                            rows.at[i], gsem.at[i]))
                    for i in range(4):
                        gd[i].wait()
                        pltpu.async_copy(rows.at[i],
                                         acc.at[dstb.at[bank, g * 4 + i]],
                                         ssem, add=True)
                return carry
            lax.fori_loop(0, ept // (BCH * K), blk, 0)
            for i in range(4):
                pltpu.make_async_copy(rows.at[i], acc.at[pl.ds(0, K)],
                                      ssem).wait()
            plsc.subcore_barrier()

            st = pl.multiple_of(s * stripe, 8)
            colm = pl.multiple_of(oslab * dc, dc)
            pltpu.sync_copy(acc.at[pl.ds(st, stripe)],
                            out_ref.at[pl.ds(st, stripe), pl.ds(colm, dc)])
            if phase + 1 < n_slab_sc:
                plsc.subcore_barrier()

    fn = pl.kernel(
        body,
        out_type=jax.ShapeDtypeStruct((n_pad, 128), jnp.float32),
        mesh=plsc.VectorSubcoreMesh(**_SC_MESH),
        compiler_params=pltpu.CompilerParams(**_SC_PARAMS),
        scratch_types=[
            pltpu.VMEM_SHARED((n_pad, dc), jnp.float32),
            pltpu.VMEM((BCH, K), jnp.int32),
            pltpu.VMEM((2, BCH, K), jnp.int32),
            pltpu.VMEM((4, K, dc), jnp.float32),
            pltpu.VMEM((ZR, dc), jnp.float32),
            pltpu.SemaphoreType.DMA((4,)),
            pltpu.SemaphoreType.DMA,
        ],
        interpret=interpret,
    )
    return fn(gflat, src2d, dst2d)


def _deg_call(dst2d, *, n_pad, e_pad, interpret=False):
    dc = 16
    stripe = n_pad // NS

    def body(dst_ref, out_ref, acc, dstb, ones_rows, zbuf):
        c = lax.axis_index("c")
        s = lax.axis_index("s")
        wid = c * NS + s

        def zb(i, carry):
            zbuf[i, pl.ds(0, L)] = jnp.zeros((L,), jnp.float32)
            return carry
        lax.fori_loop(0, ZR, zb, 0)

        def ob(i, carry):
            ones_rows[i, pl.ds(0, L)] = jnp.ones((L,), jnp.float32)
            return carry
        lax.fori_loop(0, K, ob, 0)

        def zc(z, carry):
            st = pl.multiple_of(s * stripe + z * ZR, 8)
            pltpu.sync_copy(zbuf, acc.at[pl.ds(st, ZR)])
            return carry
        lax.fori_loop(0, stripe // ZR, zc, 0)
        plsc.subcore_barrier()

        ept = e_pad // NW
        ebase = wid * ept

        def blk(b, carry):
            cb = pl.multiple_of(ebase // K + b * BCH, 8)
            pltpu.sync_copy(dst_ref.at[pl.ds(cb, BCH)], dstb)
            for r in range(BCH):
                pltpu.sync_copy(ones_rows, acc.at[dstb.at[r]], add=True)
            return carry
        lax.fori_loop(0, ept // (BCH * K), blk, 0)
        plsc.subcore_barrier()

        st = pl.multiple_of(s * stripe, 8)
        colm = pl.multiple_of(c * dc, dc)
        pltpu.sync_copy(acc.at[pl.ds(st, stripe)],
                        out_ref.at[pl.ds(st, stripe), pl.ds(colm, dc)])

    fn = pl.kernel(
        body,
        out_type=jax.ShapeDtypeStruct((n_pad, 128), jnp.float32),
        mesh=plsc.VectorSubcoreMesh(**_SC_MESH),
        compiler_params=pltpu.CompilerParams(**_SC_PARAMS),
        scratch_types=[
            pltpu.VMEM_SHARED((n_pad, dc), jnp.float32),
            pltpu.VMEM((BCH, K), jnp.int32),
            pltpu.VMEM((K, dc), jnp.float32),
            pltpu.VMEM((ZR, dc), jnp.float32),
        ],
        interpret=interpret,
    )
    return fn(dst2d)


def _pool_call(h3, bat32, *, n_pad, g_pad, interpret=False):
    dc = 128
    npt = n_pad // NW
    nj = npt // 32
    gs = g_pad // NS

    def body(h3_ref, bat_ref, out_ref, pacc, rbuf, batb, zbuf):
        c = lax.axis_index("c")
        s = lax.axis_index("s")
        wid = c * NS + s

        def zb(i, carry):
            for j in range(dc // L):
                zbuf[i, pl.ds(j * L, L)] = jnp.zeros((L,), jnp.float32)
            return carry
        lax.fori_loop(0, gs, zb, 0)
        st = pl.multiple_of(s * gs, 8)
        pltpu.sync_copy(zbuf, pacc.at[pl.ds(st, gs)])
        plsc.subcore_barrier()

        pltpu.sync_copy(bat_ref.at[wid], batb)

        def jl(j, carry):
            hb = pl.multiple_of(wid * npt + j * 32, 8)
            pltpu.sync_copy(h3_ref.at[pl.ds(hb, 32)], rbuf)
            pltpu.sync_copy(rbuf, pacc.at[batb.at[j]], add=True)
            return carry
        lax.fori_loop(0, nj, jl, 0)
        plsc.subcore_barrier()

        pltpu.sync_copy(pacc.at[pl.ds(st, gs)],
                        out_ref.at[c, pl.ds(st, gs)])

    fn = pl.kernel(
        body,
        out_type=jax.ShapeDtypeStruct((NC, g_pad, 128), jnp.float32),
        mesh=plsc.VectorSubcoreMesh(**_SC_MESH),
        compiler_params=pltpu.CompilerParams(**_SC_PARAMS),
        scratch_types=[
            pltpu.VMEM_SHARED((g_pad, dc), jnp.float32),
            pltpu.VMEM((32, dc), jnp.float32),
            pltpu.VMEM((nj, 32), jnp.int32),
            pltpu.VMEM((gs, dc), jnp.float32),
        ],
        interpret=interpret,
    )
    return fn(h3, bat32)


def _tc0(D, xp, *, n_pad, interpret=False):
    def body(d_ref, xp_ref, g0_ref):
        deg = d_ref[:, 0:1] + d_ref[:, 16:17] + 1.0
        di = lax.rsqrt(deg)
        g16 = di * xp_ref[...]
        nrows = g16.shape[0]
        g0_ref[...] = jnp.concatenate(
            [g16[:, :3], di, jnp.zeros((nrows, 124), jnp.float32)], axis=1)

    return pl.pallas_call(
        body,
        grid=(n_pad // NB,),
        in_specs=[pl.BlockSpec((NB, 128), lambda i: (i, 0)),
                  pl.BlockSpec((NB, 16), lambda i: (i, 0))],
        out_specs=pl.BlockSpec((NB, 128), lambda i: (i, 0)),
        out_shape=jax.ShapeDtypeStruct((n_pad, 128), jnp.float32),
        interpret=interpret,
    )(D, xp)


def _tc1(A0, g0, W1p, b1, *, n_pad, interpret=False):
    def body(a_ref, g_ref, w_ref, b_ref, o_ref):
        di = g_ref[:, 3:4]
        agg = a_ref[:, :16] + a_ref[:, 16:32]
        z = di * (agg + g_ref[:, :16])
        h = jnp.dot(z.astype(jnp.bfloat16), w_ref[...].astype(jnp.bfloat16),
                    preferred_element_type=jnp.float32)
        h = jnp.maximum(h + b_ref[...], 0.0)
        o_ref[...] = di * h

    return pl.pallas_call(
        body,
        grid=(n_pad // NB,),
        in_specs=[pl.BlockSpec((NB, 128), lambda i: (i, 0)),
                  pl.BlockSpec((NB, 128), lambda i: (i, 0)),
                  pl.BlockSpec((16, 128), lambda i: (0, 0)),
                  pl.BlockSpec((1, 128), lambda i: (0, 0))],
        out_specs=pl.BlockSpec((NB, 128), lambda i: (i, 0)),
        out_shape=jax.ShapeDtypeStruct((n_pad, 128), jnp.float32),
        interpret=interpret,
    )(A0, g0, W1p, b1)


def _tc2(A1, g1, g0, W2, b2, W3, *, n_pad, interpret=False):
    def body(a_ref, g_ref, g0_ref, w2_ref, b2_ref, w3_ref, o_ref):
        di = g0_ref[:, 3:4]
        z = di * (a_ref[...] + g_ref[...])
        h2 = jnp.dot(z.astype(jnp.bfloat16), w2_ref[...].astype(jnp.bfloat16),
                     preferred_element_type=jnp.float32)
        h2 = jnp.maximum(h2 + b2_ref[...], 0.0)
        t = jnp.dot(h2.astype(jnp.bfloat16), w3_ref[...].astype(jnp.bfloat16),
                    preferred_element_type=jnp.float32)
        nrows = t.shape[0]
        o_ref[...] = jnp.concatenate(
            [di * t, di, jnp.zeros((nrows, 63), jnp.float32)], axis=1)

    return pl.pallas_call(
        body,
        grid=(n_pad // NB,),
        in_specs=[pl.BlockSpec((NB, 128), lambda i: (i, 0)),
                  pl.BlockSpec((NB, 128), lambda i: (i, 0)),
                  pl.BlockSpec((NB, 128), lambda i: (i, 0)),
                  pl.BlockSpec((128, 128), lambda i: (0, 0)),
                  pl.BlockSpec((1, 128), lambda i: (0, 0)),
                  pl.BlockSpec((128, 64), lambda i: (0, 0))],
        out_specs=pl.BlockSpec((NB, 128), lambda i: (i, 0)),
        out_shape=jax.ShapeDtypeStruct((n_pad, 128), jnp.float32),
        interpret=interpret,
    )(A1, g1, g0, W2, b2, W3)


def _tc3(A2, g2, b3, *, n_pad, interpret=False):
    def body(a_ref, g_ref, b_ref, o_ref):
        di = g_ref[:, 64:65]
        z = a_ref[:, :64] + g_ref[:, :64]
        h3 = jnp.maximum(di * z + b_ref[...], 0.0)
        nrows = h3.shape[0]
        o_ref[...] = jnp.concatenate(
            [h3, jnp.ones((nrows, 1), jnp.float32),
             jnp.zeros((nrows, 63), jnp.float32)], axis=1)

    return pl.pallas_call(
        body,
        grid=(n_pad // NB,),
        in_specs=[pl.BlockSpec((NB, 128), lambda i: (i, 0)),
                  pl.BlockSpec((NB, 128), lambda i: (i, 0)),
                  pl.BlockSpec((1, 64), lambda i: (0, 0))],
        out_specs=pl.BlockSpec((NB, 128), lambda i: (i, 0)),
        out_shape=jax.ShapeDtypeStruct((n_pad, 128), jnp.float32),
        interpret=interpret,
    )(A2, g2, b3)


def _tc4(P, Wl, bl, *, g_pad, interpret=False):
    def body(p_ref, wl_ref, bl_ref, o_ref):
        pp = p_ref[0] + p_ref[1]
        sm = pp[:, :64]
        cnt = pp[:, 64:65]
        y = jnp.dot(sm.astype(jnp.bfloat16), wl_ref[...].astype(jnp.bfloat16),
                    preferred_element_type=jnp.float32)
        o_ref[...] = y / jnp.maximum(cnt, 1.0) + bl_ref[...]

    return pl.pallas_call(
        body,
        out_shape=jax.ShapeDtypeStruct((g_pad, 1), jnp.float32),
        interpret=interpret,
    )(P, Wl, bl)


def _run(x, edge_index, batch, W1, b1, W2, b2, W3, b3, Wl, bl,
         interpret=False):
    f32 = jnp.float32
    N = x.shape[0]
    E = edge_index.shape[1]
    G = 512

    n_pad = _rup(N + 1, 7168)
    e_pad = _rup(E, NW * BCH * K)
    g_pad = _rup(G + 1, 128)

    src = edge_index[0].astype(jnp.int32)
    dst = edge_index[1].astype(jnp.int32)
    src_p = jnp.concatenate([src, jnp.zeros((e_pad - E,), jnp.int32)])
    dst_p = jnp.concatenate([dst, jnp.full((e_pad - E,), N, jnp.int32)])
    src2d = src_p.reshape(e_pad // K, K)
    dst2d = dst_p.reshape(e_pad // K, K)
    bat = batch.astype(jnp.int32)
    bat_p = jnp.concatenate([bat, jnp.full((n_pad - N,), G, jnp.int32)])
    bat32 = bat_p.reshape(NW, (n_pad // NW) // 32, 32)
    xp = jnp.zeros((n_pad, 16), f32).at[:N, :3].set(x)
    W1p = jnp.zeros((16, 128), f32).at[:3].set(W1)
    b1r = b1.reshape(1, 128)
    b2r = b2.reshape(1, 128)
    b3r = b3.reshape(1, 64)
    blr = bl.reshape(1, 1)
    wlr = Wl.reshape(64, 1)

    D = _deg_call(dst2d, n_pad=n_pad, e_pad=e_pad, interpret=interpret)
    g0 = _tc0(D, xp, n_pad=n_pad, interpret=interpret)
    A0 = _agg_call(g0.reshape(8 * n_pad, 16), src2d, dst2d, n_pad=n_pad,
                   e_pad=e_pad, dc=16, n_slab_sc=1, split_edges=True,
                   interpret=interpret)
    g1 = _tc1(A0, g0, W1p, b1r, n_pad=n_pad, interpret=interpret)
    A1 = _agg_call(g1.reshape(4 * n_pad, 32), src2d, dst2d, n_pad=n_pad,
                   e_pad=e_pad, dc=32, n_slab_sc=2, split_edges=False,
                   interpret=interpret)
    g2 = _tc2(A1, g1, g0, W2, b2r, W3, n_pad=n_pad, interpret=interpret)
    A2 = _agg_call(g2.reshape(4 * n_pad, 32), src2d, dst2d, n_pad=n_pad,
                   e_pad=e_pad, dc=32, n_slab_sc=1, split_edges=False,
                   interpret=interpret)
    h3p = _tc3(A2, g2, b3r, n_pad=n_pad, interpret=interpret)
    P = _pool_call(h3p, bat32, n_pad=n_pad, g_pad=g_pad, interpret=interpret)
    out = _tc4(P, wlr, blr, g_pad=g_pad, interpret=interpret)
    return out[:G]


def kernel(x, edge_index, batch, W1, b1, W2, b2, W3, b3, Wl, bl):
    return _run(x, edge_index, batch, W1, b1, W2, b2, W3, b3, Wl, bl)
